# baseline (speedup 1.0000x reference)
import functools

import jax
import jax.numpy as jnp
from jax import lax
from jax.experimental import pallas as pl
from jax.experimental.pallas import tpu as pltpu
from jax.experimental.pallas import tpu_sc as plsc

N = 10000
E = 320000
D = 128

NC = 2
NS = 16
NW = NC * NS
EPW = E // NW
CH = 80
NCHUNK = 128
EPWP = NCHUNK * CH
NPAD = 10240
RCH = NPAD // (NS * CH)

_mesh = plsc.VectorSubcoreMesh(
    core_axis_name="c", subcore_axis_name="s", num_cores=NC, num_subcores=NS
)


_AGRP = 8
_ANG = NCHUNK // _AGRP


@functools.partial(
    pl.kernel,
    out_type=jax.ShapeDtypeStruct((NC, NPAD, 16), jnp.float32),
    mesh=_mesh,
    scratch_types=[
        pltpu.VMEM((NCHUNK, CH), jnp.int32),
        pltpu.VMEM((CH, 16), jnp.float32),
        pltpu.VMEM((CH, 16), jnp.float32),
        pltpu.VMEM_SHARED((NPAD, 16), jnp.float32),
        pltpu.SemaphoreType.DMA,
        pltpu.SemaphoreType.DMA,
        pltpu.SemaphoreType.DMA,
    ],
)
def _deg_kernel(dst_hbm, out_hbm, didx_v, ones_v, zeros_v, hist_sh, semi, sema, semw):
    c = lax.axis_index("c")
    s = lax.axis_index("s")
    wid = s * NC + c

    pltpu.async_copy(dst_hbm.at[wid], didx_v, semi)

    @pl.loop(0, CH)
    def _fill(r):
        ones_v[r] = jnp.ones((16,), jnp.float32)
        zeros_v[r] = jnp.zeros((16,), jnp.float32)

    @pl.loop(0, RCH)
    def _zero(k):
        pltpu.sync_copy(zeros_v, hist_sh.at[pl.ds((s * RCH + k) * CH, CH)])

    pltpu.make_async_copy(dst_hbm.at[wid], didx_v, semi).wait()
    plsc.subcore_barrier()

    def _add_start(k):
        pltpu.async_copy(ones_v, hist_sh.at[didx_v.at[k]], sema, add=True)

    def _add_wait():
        pltpu.make_async_copy(ones_v, hist_sh.at[didx_v.at[0]], sema).wait()

    for j in range(_AGRP):
        _add_start(j)

    @pl.loop(0, _ANG - 1)
    def _accum(i):
        for j in range(_AGRP):
            _add_start((i + 1) * _AGRP + j)
        for j in range(_AGRP):
            _add_wait()

    for j in range(_AGRP):
        _add_wait()

    plsc.subcore_barrier()

    @pl.loop(0, RCH)
    def _writeback(k):
        r0 = (s * RCH + k) * CH
        pltpu.sync_copy(hist_sh.at[pl.ds(r0, CH)], zeros_v)
        pltpu.sync_copy(zeros_v, out_hbm.at[c, pl.ds(r0, CH)])


_RB = 2000


def _lin_body(x_ref, m_ref, h0_ref, h1_ref, w_ref, g_ref, s_ref):
    deg = 1.0 + h0_ref[...] + h1_ref[...]
    sv = m_ref[...] * lax.rsqrt(deg)
    s_ref[...] = sv
    g_ref[...] = jnp.dot(
        x_ref[...] * sv, w_ref[...], preferred_element_type=jnp.float32
    )


_linear = pl.pallas_call(
    _lin_body,
    grid=(N // _RB,),
    in_specs=[
        pl.BlockSpec((_RB, D), lambda i: (i, 0)),
        pl.BlockSpec((_RB, 1), lambda i: (i, 0)),
        pl.BlockSpec((_RB, 1), lambda i: (i, 0)),
        pl.BlockSpec((_RB, 1), lambda i: (i, 0)),
        pl.BlockSpec((D, D), lambda i: (0, 0)),
    ],
    out_specs=[
        pl.BlockSpec((_RB, D), lambda i: (i, 0)),
        pl.BlockSpec((_RB, 1), lambda i: (i, 0)),
    ],
    out_shape=[
        jax.ShapeDtypeStruct((N, D), jnp.float32),
        jax.ShapeDtypeStruct((N, 1), jnp.float32),
    ],
)


@functools.partial(
    pl.kernel,
    out_type=jax.ShapeDtypeStruct((NC, NPAD, D), jnp.float32),
    mesh=_mesh,
    scratch_types=[
        pltpu.VMEM((CH,), jnp.int32),
        pltpu.VMEM((CH,), jnp.int32),
        pltpu.VMEM((CH, D), jnp.float32),
        pltpu.VMEM_SHARED((NPAD, D), jnp.float32),
        pltpu.SemaphoreType.DMA,
    ],
)
def _edge_kernel(
    src_hbm, dst_hbm, g_hbm, out_hbm,
    sidx_v, didx_v, rows_v, acc_sh, semg,
):
    c = lax.axis_index("c")
    s = lax.axis_index("s")
    wid = s * NC + c

    @pl.loop(0, CH)
    def _zero_rows(r):
        for j in range(D // 16):
            rows_v[r, pl.ds(j * 16, 16)] = jnp.zeros((16,), jnp.float32)

    @pl.loop(0, RCH)
    def _zero_acc(k):
        pltpu.sync_copy(rows_v, acc_sh.at[pl.ds((s * RCH + k) * CH, CH)])

    plsc.subcore_barrier()

    @pl.loop(0, NCHUNK)
    def _accum(i):
        pltpu.sync_copy(src_hbm.at[wid, i], sidx_v)
        pltpu.sync_copy(dst_hbm.at[wid, i], didx_v)
        pltpu.async_copy(g_hbm.at[sidx_v], rows_v, semg).wait()
        pltpu.sync_copy(rows_v, acc_sh.at[didx_v], add=True)

    plsc.subcore_barrier()

    @pl.loop(0, RCH)
    def _writeback(k):
        r0 = (s * RCH + k) * CH
        pltpu.sync_copy(acc_sh.at[pl.ds(r0, CH)], rows_v)
        pltpu.sync_copy(rows_v, out_hbm.at[c, pl.ds(r0, CH)])


def _fin_body(p_ref, g_ref, s_ref, m_ref, b_ref, o_ref):
    acc = p_ref[0] + p_ref[1] + g_ref[...]
    o_ref[...] = s_ref[...] * acc + m_ref[...] * b_ref[...]


_final = pl.pallas_call(
    _fin_body,
    grid=(N // _RB,),
    in_specs=[
        pl.BlockSpec((NC, _RB, D), lambda i: (0, i, 0)),
        pl.BlockSpec((_RB, D), lambda i: (i, 0)),
        pl.BlockSpec((_RB, 1), lambda i: (i, 0)),
        pl.BlockSpec((_RB, 1), lambda i: (i, 0)),
        pl.BlockSpec((1, D), lambda i: (0, 0)),
    ],
    out_specs=pl.BlockSpec((_RB, D), lambda i: (i, 0)),
    out_shape=jax.ShapeDtypeStruct((N, D), jnp.float32),
)


def kernel(x, edge_index, mask, W, b):
    pad_src = jnp.zeros((NW, EPWP - EPW), jnp.int32)
    pad_dst = jnp.full((NW, EPWP - EPW), NPAD - 1, jnp.int32)
    src = jnp.concatenate(
        [edge_index[0].reshape(NW, EPW), pad_src], axis=1
    ).reshape(NW, NCHUNK, CH)
    dst = jnp.concatenate(
        [edge_index[1].reshape(NW, EPW), pad_dst], axis=1
    ).reshape(NW, NCHUNK, CH)
    mask_f = mask.astype(jnp.float32).reshape(N, 1)
    hist = _deg_kernel(dst)
    h0 = hist[0, :N, 0:1]
    h1 = hist[1, :N, 0:1]
    g, sv = _linear(x, mask_f, h0, h1, W)
    p = _edge_kernel(src, dst, g)
    return _final(p, g, sv, mask_f, b.reshape(1, D))


# speedup vs baseline: 1.0116x; 1.0116x over previous
import functools

import jax
import jax.numpy as jnp
from jax import lax
from jax.experimental import pallas as pl
from jax.experimental.pallas import tpu as pltpu
from jax.experimental.pallas import tpu_sc as plsc

N = 10000
E = 320000
D = 128

NC = 2
NS = 16
NW = NC * NS
EPW = E // NW
CH = 80
NCHUNK = 128
EPWP = NCHUNK * CH
NPAD = 10240
RCH = NPAD // (NS * CH)

_mesh = plsc.VectorSubcoreMesh(
    core_axis_name="c", subcore_axis_name="s", num_cores=NC, num_subcores=NS
)


_AGRP = 8
_ANG = NCHUNK // _AGRP


@functools.partial(
    pl.kernel,
    out_type=jax.ShapeDtypeStruct((NC, NPAD, 16), jnp.float32),
    mesh=_mesh,
    scratch_types=[
        pltpu.VMEM((NCHUNK, CH), jnp.int32),
        pltpu.VMEM((CH, 16), jnp.float32),
        pltpu.VMEM((CH, 16), jnp.float32),
        pltpu.VMEM_SHARED((NPAD, 16), jnp.float32),
        pltpu.SemaphoreType.DMA,
        pltpu.SemaphoreType.DMA,
        pltpu.SemaphoreType.DMA,
    ],
)
def _deg_kernel(dst_hbm, out_hbm, didx_v, ones_v, zeros_v, hist_sh, semi, sema, semw):
    c = lax.axis_index("c")
    s = lax.axis_index("s")
    wid = s * NC + c

    pltpu.async_copy(dst_hbm.at[wid], didx_v, semi)

    @pl.loop(0, CH)
    def _fill(r):
        ones_v[r] = jnp.ones((16,), jnp.float32)
        zeros_v[r] = jnp.zeros((16,), jnp.float32)

    @pl.loop(0, RCH)
    def _zero(k):
        pltpu.sync_copy(zeros_v, hist_sh.at[pl.ds((s * RCH + k) * CH, CH)])

    pltpu.make_async_copy(dst_hbm.at[wid], didx_v, semi).wait()
    plsc.subcore_barrier()

    def _add_start(k):
        pltpu.async_copy(ones_v, hist_sh.at[didx_v.at[k]], sema, add=True)

    def _add_wait():
        pltpu.make_async_copy(ones_v, hist_sh.at[didx_v.at[0]], sema).wait()

    for j in range(_AGRP):
        _add_start(j)

    @pl.loop(0, _ANG - 1)
    def _accum(i):
        for j in range(_AGRP):
            _add_start((i + 1) * _AGRP + j)
        for j in range(_AGRP):
            _add_wait()

    for j in range(_AGRP):
        _add_wait()

    plsc.subcore_barrier()

    @pl.loop(0, RCH)
    def _writeback(k):
        r0 = (s * RCH + k) * CH
        pltpu.sync_copy(hist_sh.at[pl.ds(r0, CH)], zeros_v)
        pltpu.sync_copy(zeros_v, out_hbm.at[c, pl.ds(r0, CH)])


_RB = 2000


def _lin_body(x_ref, m_ref, h0_ref, h1_ref, w_ref, g_ref, s_ref):
    deg = 1.0 + h0_ref[...] + h1_ref[...]
    sv = m_ref[...] * lax.rsqrt(deg)
    s_ref[...] = sv
    g_ref[...] = jnp.dot(
        x_ref[...] * sv, w_ref[...], preferred_element_type=jnp.float32
    )


_linear = pl.pallas_call(
    _lin_body,
    grid=(N // _RB,),
    in_specs=[
        pl.BlockSpec((_RB, D), lambda i: (i, 0)),
        pl.BlockSpec((_RB, 1), lambda i: (i, 0)),
        pl.BlockSpec((_RB, 1), lambda i: (i, 0)),
        pl.BlockSpec((_RB, 1), lambda i: (i, 0)),
        pl.BlockSpec((D, D), lambda i: (0, 0)),
    ],
    out_specs=[
        pl.BlockSpec((_RB, D), lambda i: (i, 0)),
        pl.BlockSpec((_RB, 1), lambda i: (i, 0)),
    ],
    out_shape=[
        jax.ShapeDtypeStruct((N, D), jnp.float32),
        jax.ShapeDtypeStruct((N, 1), jnp.float32),
    ],
)


@functools.partial(
    pl.kernel,
    out_type=jax.ShapeDtypeStruct((NC, NPAD, D), jnp.float32),
    mesh=_mesh,
    scratch_types=[
        pltpu.VMEM((CH,), jnp.int32),
        pltpu.VMEM((CH,), jnp.int32),
        pltpu.VMEM((CH, D), jnp.float32),
        pltpu.VMEM_SHARED((NPAD, D), jnp.float32),
        pltpu.SemaphoreType.DMA,
    ],
)
def _edge_kernel(
    src_hbm, dst_hbm, g_hbm, out_hbm,
    sidx_v, didx_v, rows_v, acc_sh, semg,
):
    c = lax.axis_index("c")
    s = lax.axis_index("s")
    wid = s * NC + c

    @pl.loop(0, CH)
    def _zero_rows(r):
        for j in range(D // 16):
            rows_v[r, pl.ds(j * 16, 16)] = jnp.zeros((16,), jnp.float32)

    @pl.loop(0, RCH)
    def _zero_acc(k):
        pltpu.sync_copy(rows_v, acc_sh.at[pl.ds((s * RCH + k) * CH, CH)])

    plsc.subcore_barrier()

    base = wid * EPWP

    @pl.loop(0, NCHUNK)
    def _accum(i):
        off = base + i * CH
        pltpu.sync_copy(src_hbm.at[pl.ds(off, CH)], sidx_v)
        pltpu.sync_copy(dst_hbm.at[pl.ds(off, CH)], didx_v)
        pltpu.async_copy(g_hbm.at[sidx_v], rows_v, semg).wait()
        pltpu.sync_copy(rows_v, acc_sh.at[didx_v], add=True)

    plsc.subcore_barrier()

    @pl.loop(0, RCH)
    def _writeback(k):
        r0 = (s * RCH + k) * CH
        pltpu.sync_copy(acc_sh.at[pl.ds(r0, CH)], rows_v)
        pltpu.sync_copy(rows_v, out_hbm.at[c, pl.ds(r0, CH)])


def _fin_body(p_ref, g_ref, s_ref, m_ref, b_ref, o_ref):
    acc = p_ref[0] + p_ref[1] + g_ref[...]
    o_ref[...] = s_ref[...] * acc + m_ref[...] * b_ref[...]


_final = pl.pallas_call(
    _fin_body,
    grid=(N // _RB,),
    in_specs=[
        pl.BlockSpec((NC, _RB, D), lambda i: (0, i, 0)),
        pl.BlockSpec((_RB, D), lambda i: (i, 0)),
        pl.BlockSpec((_RB, 1), lambda i: (i, 0)),
        pl.BlockSpec((_RB, 1), lambda i: (i, 0)),
        pl.BlockSpec((1, D), lambda i: (0, 0)),
    ],
    out_specs=pl.BlockSpec((_RB, D), lambda i: (i, 0)),
    out_shape=jax.ShapeDtypeStruct((N, D), jnp.float32),
)


def kernel(x, edge_index, mask, W, b):
    pad_src = jnp.zeros((NW, EPWP - EPW), jnp.int32)
    pad_dst = jnp.full((NW, EPWP - EPW), NPAD - 1, jnp.int32)
    src = jnp.concatenate(
        [edge_index[0].reshape(NW, EPW), pad_src], axis=1
    ).reshape(NW, NCHUNK, CH)
    dst = jnp.concatenate(
        [edge_index[1].reshape(NW, EPW), pad_dst], axis=1
    ).reshape(NW, NCHUNK, CH)
    mask_f = mask.astype(jnp.float32).reshape(N, 1)
    hist = _deg_kernel(dst)
    h0 = hist[0, :N, 0:1]
    h1 = hist[1, :N, 0:1]
    g, sv = _linear(x, mask_f, h0, h1, W)
    p = _edge_kernel(src.reshape(NW * EPWP), dst.reshape(NW * EPWP), g)
    return _final(p, g, sv, mask_f, b.reshape(1, D))


# speedup vs baseline: 1.7144x; 1.6947x over previous
import functools

import jax
import jax.numpy as jnp
from jax import lax
from jax.experimental import pallas as pl
from jax.experimental.pallas import tpu as pltpu
from jax.experimental.pallas import tpu_sc as plsc

N = 10000
E = 320000
D = 128

NC = 2
NS = 16
NW = NC * NS
EPW = E // NW
CH = 80
NCHUNK = 128
EPWP = NCHUNK * CH
NPAD = 10240
RCH = NPAD // (NS * CH)

_mesh = plsc.VectorSubcoreMesh(
    core_axis_name="c", subcore_axis_name="s", num_cores=NC, num_subcores=NS
)


_AGRP = 8
_ANG = NCHUNK // _AGRP


@functools.partial(
    pl.kernel,
    out_type=jax.ShapeDtypeStruct((NC, NPAD, 16), jnp.float32),
    mesh=_mesh,
    scratch_types=[
        pltpu.VMEM((NCHUNK, CH), jnp.int32),
        pltpu.VMEM((CH, 16), jnp.float32),
        pltpu.VMEM((CH, 16), jnp.float32),
        pltpu.VMEM_SHARED((NPAD, 16), jnp.float32),
        pltpu.SemaphoreType.DMA,
        pltpu.SemaphoreType.DMA,
        pltpu.SemaphoreType.DMA,
    ],
)
def _deg_kernel(dst_hbm, out_hbm, didx_v, ones_v, zeros_v, hist_sh, semi, sema, semw):
    c = lax.axis_index("c")
    s = lax.axis_index("s")
    wid = s * NC + c

    pltpu.async_copy(dst_hbm.at[wid], didx_v, semi)

    @pl.loop(0, CH)
    def _fill(r):
        ones_v[r] = jnp.ones((16,), jnp.float32)
        zeros_v[r] = jnp.zeros((16,), jnp.float32)

    @pl.loop(0, RCH)
    def _zero(k):
        pltpu.sync_copy(zeros_v, hist_sh.at[pl.ds((s * RCH + k) * CH, CH)])

    pltpu.make_async_copy(dst_hbm.at[wid], didx_v, semi).wait()
    plsc.subcore_barrier()

    def _add_start(k):
        pltpu.async_copy(ones_v, hist_sh.at[didx_v.at[k]], sema, add=True)

    def _add_wait():
        pltpu.make_async_copy(ones_v, hist_sh.at[didx_v.at[0]], sema).wait()

    for j in range(_AGRP):
        _add_start(j)

    @pl.loop(0, _ANG - 1)
    def _accum(i):
        for j in range(_AGRP):
            _add_start((i + 1) * _AGRP + j)
        for j in range(_AGRP):
            _add_wait()

    for j in range(_AGRP):
        _add_wait()

    plsc.subcore_barrier()

    @pl.loop(0, RCH)
    def _writeback(k):
        r0 = (s * RCH + k) * CH
        pltpu.sync_copy(hist_sh.at[pl.ds(r0, CH)], zeros_v)
        pltpu.sync_copy(zeros_v, out_hbm.at[c, pl.ds(r0, CH)])


_RB = 2000


def _lin_body(x_ref, m_ref, h0_ref, h1_ref, w_ref, g_ref, s_ref):
    deg = 1.0 + h0_ref[...] + h1_ref[...]
    sv = m_ref[...] * lax.rsqrt(deg)
    s_ref[...] = sv
    g_ref[...] = jnp.dot(
        x_ref[...] * sv, w_ref[...], preferred_element_type=jnp.float32
    )


_linear = pl.pallas_call(
    _lin_body,
    grid=(N // _RB,),
    in_specs=[
        pl.BlockSpec((_RB, D), lambda i: (i, 0)),
        pl.BlockSpec((_RB, 1), lambda i: (i, 0)),
        pl.BlockSpec((_RB, 1), lambda i: (i, 0)),
        pl.BlockSpec((_RB, 1), lambda i: (i, 0)),
        pl.BlockSpec((D, D), lambda i: (0, 0)),
    ],
    out_specs=[
        pl.BlockSpec((_RB, D), lambda i: (i, 0)),
        pl.BlockSpec((_RB, 1), lambda i: (i, 0)),
    ],
    out_shape=[
        jax.ShapeDtypeStruct((N, D), jnp.float32),
        jax.ShapeDtypeStruct((N, 1), jnp.float32),
    ],
)


@functools.partial(
    pl.kernel,
    out_type=jax.ShapeDtypeStruct((NC, NPAD, D), jnp.float32),
    mesh=_mesh,
    scratch_types=[
        pltpu.VMEM((CH,), jnp.int32),
        pltpu.VMEM((CH,), jnp.int32),
        pltpu.VMEM((CH, D), jnp.float32),
        pltpu.VMEM_SHARED((NPAD, D), jnp.float32),
        pltpu.SemaphoreType.DMA,
    ],
)
def _edge_kernel(
    src_hbm, dst_hbm, g_hbm, out_hbm,
    sidx_v, didx_v, rows_v, acc_sh, semg,
):
    c = lax.axis_index("c")
    s = lax.axis_index("s")
    wid = s * NC + c

    @pl.loop(0, CH)
    def _zero_rows(r):
        for j in range(D // 16):
            rows_v[r, pl.ds(j * 16, 16)] = jnp.zeros((16,), jnp.float32)

    @pl.loop(0, RCH)
    def _zero_acc(k):
        pltpu.sync_copy(rows_v, acc_sh.at[pl.ds((s * RCH + k) * CH, CH)])

    plsc.subcore_barrier()

    base = wid * EPWP

    @pl.loop(0, NCHUNK)
    def _accum(i):
        off = base + i * CH
        pltpu.sync_copy(src_hbm.at[pl.ds(off, CH)], sidx_v)
        pltpu.sync_copy(dst_hbm.at[pl.ds(off, CH)], didx_v)
        pltpu.async_copy(g_hbm.at[sidx_v], rows_v, semg).wait()
        pltpu.sync_copy(rows_v, acc_sh.at[didx_v], add=True)

    plsc.subcore_barrier()

    @pl.loop(0, RCH)
    def _writeback(k):
        r0 = (s * RCH + k) * CH
        pltpu.sync_copy(acc_sh.at[pl.ds(r0, CH)], rows_v)
        pltpu.sync_copy(rows_v, out_hbm.at[c, pl.ds(r0, CH)])


def _fin_body(p_ref, g_ref, s_ref, m_ref, b_ref, o_ref):
    acc = p_ref[0] + p_ref[1] + g_ref[...]
    o_ref[...] = s_ref[...] * acc + m_ref[...] * b_ref[...]


_final = pl.pallas_call(
    _fin_body,
    grid=(N // _RB,),
    in_specs=[
        pl.BlockSpec((NC, _RB, D), lambda i: (0, i, 0)),
        pl.BlockSpec((_RB, D), lambda i: (i, 0)),
        pl.BlockSpec((_RB, 1), lambda i: (i, 0)),
        pl.BlockSpec((_RB, 1), lambda i: (i, 0)),
        pl.BlockSpec((1, D), lambda i: (0, 0)),
    ],
    out_specs=pl.BlockSpec((_RB, D), lambda i: (i, 0)),
    out_shape=jax.ShapeDtypeStruct((N, D), jnp.float32),
)


def kernel(x, edge_index, mask, W, b):
    pad_src = jnp.broadcast_to(
        jnp.arange(0, EPWP - EPW, dtype=jnp.int32), (NW, EPWP - EPW)
    )
    pad_dst = jnp.broadcast_to(
        jnp.arange(N, NPAD, dtype=jnp.int32), (NW, EPWP - EPW)
    )
    src = jnp.concatenate(
        [edge_index[0].reshape(NW, EPW), pad_src], axis=1
    ).reshape(NW, NCHUNK, CH)
    dst = jnp.concatenate(
        [edge_index[1].reshape(NW, EPW), pad_dst], axis=1
    ).reshape(NW, NCHUNK, CH)
    mask_f = mask.astype(jnp.float32).reshape(N, 1)
    hist = _deg_kernel(dst)
    h0 = hist[0, :N, 0:1]
    h1 = hist[1, :N, 0:1]
    g, sv = _linear(x, mask_f, h0, h1, W)
    p = _edge_kernel(src.reshape(NW * EPWP), dst.reshape(NW * EPWP), g)
    return _final(p, g, sv, mask_f, b.reshape(1, D))
